# Initial kernel scaffold; baseline (speedup 1.0000x reference)
import jax
import jax.numpy as jnp
from jax.experimental import pallas as pl


def kernel(x, edge_index, W1, b1, W2, b2, Wf1, bf1, Wf2, bf2):
    raise NotImplementedError("write your pallas kernel here")


# speedup vs baseline: 24.3408x; 24.3408x over previous
import functools

import jax
import jax.numpy as jnp
from jax import lax
from jax.experimental import pallas as pl
from jax.experimental.pallas import tpu as pltpu
from jax.experimental.pallas import tpu_sc as plsc

NC = 2
NS = 16
LANES = 16
CH = 128
WIN = 1024
NCH = WIN // CH
RB = 4
DUM = 128


def _round8(v):
    return (v + 7) // 8 * 8


def _sc_mesh():
    return plsc.VectorSubcoreMesh(core_axis_name="c", subcore_axis_name="s")


def _compute_offsets(d_win, offs, base, half):
    lane = lax.iota(jnp.int32, LANES)
    for j in range(NCH):
        for k in range(CH // LANES):
            dd = d_win[j, pl.ds(k * LANES, LANES)]
            loc = dd - base
            inr = (dd >= base) & (loc < half)
            dum = half + ((j * (CH // LANES) + k) % 8) * LANES + lane
            offs[j, pl.ds(k * LANES, LANES)] = jnp.where(inr, loc, dum)


def _make_deg_kernel(n, half, nwin):
    sh = _round8((half + DUM + NS - 1) // NS) * NS
    zb = sh // NS
    cw = _round8((half + NS - 1) // NS)
    last = half - (NS - 1) * cw
    t_rows = nwin * NCH

    @functools.partial(
        pl.kernel,
        out_type=jax.ShapeDtypeStruct((n,), jnp.float32),
        mesh=_sc_mesh(),
        compiler_params=pltpu.CompilerParams(use_tc_tiling_on_sc=False),
        scratch_types=[
            pltpu.VMEM((NCH, CH), jnp.int32),
            pltpu.VMEM((NCH, CH), jnp.int32),
            pltpu.VMEM((CH,), jnp.float32),
            pltpu.VMEM((zb,), jnp.float32),
            pltpu.VMEM_SHARED((sh,), jnp.float32),
        ],
    )
    def deg_kernel(d2_hbm, deg_out, d_win, offs, ones, zbuf, deg_sh):
        cid = lax.axis_index("c")
        sid = lax.axis_index("s")
        base = cid * half

        def z_body(i, _):
            zbuf[pl.ds(i * LANES, LANES)] = jnp.zeros((LANES,), jnp.float32)
            return 0

        lax.fori_loop(0, zb // LANES, z_body, 0)

        def o_body(i, _):
            ones[pl.ds(i * LANES, LANES)] = jnp.ones((LANES,), jnp.float32)
            return 0

        lax.fori_loop(0, CH // LANES, o_body, 0)
        pltpu.sync_copy(zbuf, deg_sh.at[pl.ds(pl.multiple_of(sid * zb, 8),
                                              zb)])
        plsc.subcore_barrier()

        def win_body(w, _):
            row0 = pl.multiple_of(sid * t_rows + w * NCH, 8)
            pltpu.sync_copy(d2_hbm.at[pl.ds(row0, NCH)], d_win)
            _compute_offsets(d_win, offs, base, half)
            for j in range(NCH):
                pltpu.sync_copy(ones, deg_sh.at[offs.at[j]], add=True)
            return 0

        lax.fori_loop(0, nwin, win_body, 0)
        plsc.subcore_barrier()

        @pl.when(sid < NS - 1)
        def _():
            off = pl.multiple_of(sid * cw, 8)
            pltpu.sync_copy(deg_sh.at[pl.ds(off, cw)], zbuf.at[pl.ds(0, cw)])
            pltpu.sync_copy(zbuf.at[pl.ds(0, cw)],
                            deg_out.at[pl.ds(pl.multiple_of(base + off, 8),
                                             cw)])

        @pl.when(sid == NS - 1)
        def _():
            pltpu.sync_copy(deg_sh.at[pl.ds((NS - 1) * cw, last)],
                            zbuf.at[pl.ds(0, last)])
            pltpu.sync_copy(
                zbuf.at[pl.ds(0, last)],
                deg_out.at[pl.ds(pl.multiple_of(base + (NS - 1) * cw, 8),
                                 last)])

    return deg_kernel


def _make_msg_kernel(n, f, half, nwin):
    cw_r = _round8((half + NS - 1) // NS)
    last_r = half - (NS - 1) * cw_r
    t_rows = nwin * NCH

    @functools.partial(
        pl.kernel,
        out_type=jax.ShapeDtypeStruct((n, f), jnp.float32),
        mesh=_sc_mesh(),
        compiler_params=pltpu.CompilerParams(use_tc_tiling_on_sc=False),
        scratch_types=[
            pltpu.VMEM((NCH, CH), jnp.int32),
            pltpu.VMEM((NCH, CH), jnp.int32),
            pltpu.VMEM((NCH, CH), jnp.int32),
            pltpu.VMEM((RB * CH, f), jnp.float32),
            pltpu.VMEM_SHARED((half + DUM, f), jnp.float32),
            pltpu.SemaphoreType.DMA,
        ],
    )
    def msg_kernel(g_hbm, s2_hbm, d2_hbm, acc_out,
                   s_win, d_win, offs, rows, acc_sh, sem):
        cid = lax.axis_index("c")
        sid = lax.axis_index("s")
        base = cid * half

        def stage_rows(off0, cnt, to_spmem):
            blk = RB * CH
            nfull, rem = divmod(cnt, blk)

            def body(q, _):
                off = pl.multiple_of(off0 + q * blk, 8)
                hoff = pl.multiple_of(base + off, 8)
                if to_spmem:
                    pltpu.sync_copy(g_hbm.at[pl.ds(hoff, blk)], rows)
                    pltpu.sync_copy(rows, acc_sh.at[pl.ds(off, blk)])
                else:
                    pltpu.sync_copy(acc_sh.at[pl.ds(off, blk)], rows)
                    pltpu.sync_copy(rows, acc_out.at[pl.ds(hoff, blk)])
                return 0

            lax.fori_loop(0, nfull, body, 0)
            if rem:
                off = pl.multiple_of(off0 + nfull * blk, 8)
                hoff = pl.multiple_of(base + off, 8)
                if to_spmem:
                    pltpu.sync_copy(g_hbm.at[pl.ds(hoff, rem)],
                                    rows.at[pl.ds(0, rem)])
                    pltpu.sync_copy(rows.at[pl.ds(0, rem)],
                                    acc_sh.at[pl.ds(off, rem)])
                else:
                    pltpu.sync_copy(acc_sh.at[pl.ds(off, rem)],
                                    rows.at[pl.ds(0, rem)])
                    pltpu.sync_copy(rows.at[pl.ds(0, rem)],
                                    acc_out.at[pl.ds(hoff, rem)])

        @pl.when(sid < NS - 1)
        def _():
            stage_rows(sid * cw_r, cw_r, True)

        @pl.when(sid == NS - 1)
        def _():
            stage_rows((NS - 1) * cw_r, last_r, True)

        plsc.subcore_barrier()

        def win_body(w, _):
            row0 = pl.multiple_of(sid * t_rows + w * NCH, 8)
            pltpu.sync_copy(s2_hbm.at[pl.ds(row0, NCH)], s_win)
            pltpu.sync_copy(d2_hbm.at[pl.ds(row0, NCH)], d_win)
            _compute_offsets(d_win, offs, base, half)
            for h in range(NCH // RB):
                copies = [
                    pltpu.async_copy(g_hbm.at[s_win.at[h * RB + j]],
                                     rows.at[pl.ds(j * CH, CH)], sem)
                    for j in range(RB)
                ]
                for c in copies:
                    c.wait()
                for j in range(RB):
                    pltpu.sync_copy(rows.at[pl.ds(j * CH, CH)],
                                    acc_sh.at[offs.at[h * RB + j]], add=True)
            return 0

        lax.fori_loop(0, nwin, win_body, 0)
        plsc.subcore_barrier()

        @pl.when(sid < NS - 1)
        def _():
            stage_rows(sid * cw_r, cw_r, False)

        @pl.when(sid == NS - 1)
        def _():
            stage_rows((NS - 1) * cw_r, last_r, False)

    return msg_kernel


BN = 2000


def _tc_call(body, n, in_specs, out_specs, out_shapes):
    return pl.pallas_call(
        body,
        grid=(n // BN,),
        in_specs=in_specs,
        out_specs=out_specs,
        out_shape=out_shapes,
    )


def _row_spec(f):
    return pl.BlockSpec((BN, f), lambda i: (i, 0))


def _full_spec(a, b):
    return pl.BlockSpec((a, b), lambda i: (0, 0))


def _dense1_body(deg_ref, x_ref, w1_ref, g1_ref, dinv_ref):
    dinv = lax.rsqrt(deg_ref[...] + 1.0)
    g1_ref[...] = dinv * jnp.dot(x_ref[...], w1_ref[...],
                                 preferred_element_type=jnp.float32)
    dinv_ref[...] = dinv


def _dense2_body(acc_ref, dinv_ref, b1_ref, w2_ref, g2_ref):
    dinv = dinv_ref[...]
    h = jnp.maximum(dinv * acc_ref[...] + b1_ref[...], 0.0)
    g2_ref[...] = dinv * jnp.dot(h, w2_ref[...],
                                 preferred_element_type=jnp.float32)


def _dense3_body(acc_ref, dinv_ref, b2_ref, wf1_ref, bf1_ref, wf2_ref,
                 bf2_ref, out_ref):
    h = jnp.maximum(dinv_ref[...] * acc_ref[...] + b2_ref[...], 0.0)
    h = jnp.maximum(jnp.dot(h, wf1_ref[...],
                            preferred_element_type=jnp.float32) + bf1_ref[...],
                    0.0)
    out_ref[...] = jnp.dot(h, wf2_ref[...],
                           preferred_element_type=jnp.float32) + bf2_ref[...]


def kernel(x, edge_index, W1, b1, W2, b2, Wf1, bf1, Wf2, bf2):
    n, fin = x.shape
    e = edge_index.shape[1]
    half = n // 2
    f1 = W1.shape[1]
    f2 = W2.shape[1]
    ff1 = Wf1.shape[1]
    fout = Wf2.shape[1]

    per_round = NS * WIN
    nwin = -(-e // per_round)
    e_pad = nwin * per_round
    pad = e_pad - e
    s_pad = jnp.concatenate(
        [edge_index[0], jnp.arange(pad, dtype=jnp.int32) % n])
    d_pad = jnp.concatenate(
        [edge_index[1], jnp.full((pad,), -1, jnp.int32)])
    s2 = s_pad.reshape(e_pad // CH, CH)
    d2 = d_pad.reshape(e_pad // CH, CH)

    deg = _make_deg_kernel(n, half, nwin)(d2)
    deg2 = deg.reshape(n, 1)

    msg = _make_msg_kernel(n, f1, half, nwin)

    g1, dinv = _tc_call(
        _dense1_body, n,
        [_row_spec(1), _row_spec(fin), _full_spec(fin, f1)],
        [_row_spec(f1), _row_spec(1)],
        [jax.ShapeDtypeStruct((n, f1), jnp.float32),
         jax.ShapeDtypeStruct((n, 1), jnp.float32)],
    )(deg2, x, W1)

    acc1 = msg(g1, s2, d2)

    g2 = _tc_call(
        _dense2_body, n,
        [_row_spec(f1), _row_spec(1), _full_spec(1, f1), _full_spec(f1, f2)],
        _row_spec(f2),
        jax.ShapeDtypeStruct((n, f2), jnp.float32),
    )(acc1, dinv, b1.reshape(1, f1), W2)

    acc2 = msg(g2, s2, d2)

    out = _tc_call(
        _dense3_body, n,
        [_row_spec(f2), _row_spec(1), _full_spec(1, f2), _full_spec(f2, ff1),
         _full_spec(1, ff1), _full_spec(ff1, fout), _full_spec(1, fout)],
        _row_spec(fout),
        jax.ShapeDtypeStruct((n, fout), jnp.float32),
    )(acc2, dinv, b2.reshape(1, f2), Wf1, bf1.reshape(1, ff1), Wf2,
      bf2.reshape(1, fout))

    return out


# speedup vs baseline: 37.0393x; 1.5217x over previous
import functools

import jax
import jax.numpy as jnp
from jax import lax
from jax.experimental import pallas as pl
from jax.experimental.pallas import tpu as pltpu
from jax.experimental.pallas import tpu_sc as plsc

NC = 2
NS = 16
LANES = 16
VSZ = 256
NB = 3
DUM = 128


def _round8(v):
    return (v + 7) // 8 * 8


def _sc_mesh():
    return plsc.VectorSubcoreMesh(core_axis_name="c", subcore_axis_name="s")


def _offsets_chunk(d_win, offs, b, base, half):
    lane = lax.iota(jnp.int32, LANES)
    for k in range(VSZ // LANES):
        dd = d_win[b, pl.ds(k * LANES, LANES)]
        loc = dd - base
        inr = (dd >= base) & (loc < half)
        dum = half + (k % 8) * LANES + lane
        offs[b, pl.ds(k * LANES, LANES)] = jnp.where(inr, loc, dum)


def _make_deg_kernel(n, half, nwin):
    sh = _round8((half + DUM + NS - 1) // NS) * NS
    zb = sh // NS
    cw = _round8((half + NS - 1) // NS)
    last = half - (NS - 1) * cw
    t_edges = nwin * VSZ

    @functools.partial(
        pl.kernel,
        out_type=jax.ShapeDtypeStruct((n,), jnp.float32),
        mesh=_sc_mesh(),
        compiler_params=pltpu.CompilerParams(use_tc_tiling_on_sc=False),
        scratch_types=[
            pltpu.VMEM((NB, VSZ), jnp.int32),
            pltpu.VMEM((NB, VSZ), jnp.int32),
            pltpu.VMEM((VSZ,), jnp.float32),
            pltpu.VMEM((zb,), jnp.float32),
            pltpu.VMEM_SHARED((sh,), jnp.float32),
            pltpu.SemaphoreType.DMA((NB,)),
            pltpu.SemaphoreType.DMA((NB,)),
        ],
    )
    def deg_kernel(d1_hbm, deg_out, d_win, offs, ones, zbuf, deg_sh,
                   isem, ssem):
        cid = lax.axis_index("c")
        sid = lax.axis_index("s")
        base = cid * half

        def z_body(i, _):
            zbuf[pl.ds(i * LANES, LANES)] = jnp.zeros((LANES,), jnp.float32)
            return 0

        lax.fori_loop(0, zb // LANES, z_body, 0)

        def o_body(i, _):
            ones[pl.ds(i * LANES, LANES)] = jnp.ones((LANES,), jnp.float32)
            return 0

        lax.fori_loop(0, VSZ // LANES, o_body, 0)
        pltpu.sync_copy(zbuf, deg_sh.at[pl.ds(pl.multiple_of(sid * zb, 8),
                                              zb)])
        plsc.subcore_barrier()

        def e0_of(v):
            return pl.multiple_of(sid * t_edges + v * VSZ, 8)

        for b in range(NB):
            pltpu.sync_copy(d1_hbm.at[pl.ds(e0_of(b), VSZ)], d_win.at[b])

        def win_body(w, _):
            for b in range(NB):
                v = w * NB + b

                @pl.when(v >= NB)
                def _():
                    pltpu.make_async_copy(d1_hbm.at[pl.ds(0, VSZ)],
                                          d_win.at[b], isem.at[b]).wait()
                    pltpu.make_async_copy(ones, deg_sh.at[offs.at[b]],
                                          ssem.at[b]).wait()

                _offsets_chunk(d_win, offs, b, base, half)
                pltpu.async_copy(ones, deg_sh.at[offs.at[b]], ssem.at[b],
                                 add=True)

                @pl.when(v + NB < nwin)
                def _():
                    pltpu.async_copy(d1_hbm.at[pl.ds(e0_of(v + NB), VSZ)],
                                     d_win.at[b], isem.at[b])
            return 0

        lax.fori_loop(0, nwin // NB, win_body, 0)
        for b in range(NB):
            pltpu.make_async_copy(ones, deg_sh.at[offs.at[b]],
                                  ssem.at[b]).wait()
        plsc.subcore_barrier()

        @pl.when(sid < NS - 1)
        def _():
            off = pl.multiple_of(sid * cw, 8)
            pltpu.sync_copy(deg_sh.at[pl.ds(off, cw)], zbuf.at[pl.ds(0, cw)])
            pltpu.sync_copy(zbuf.at[pl.ds(0, cw)],
                            deg_out.at[pl.ds(pl.multiple_of(base + off, 8),
                                             cw)])

        @pl.when(sid == NS - 1)
        def _():
            pltpu.sync_copy(deg_sh.at[pl.ds((NS - 1) * cw, last)],
                            zbuf.at[pl.ds(0, last)])
            pltpu.sync_copy(
                zbuf.at[pl.ds(0, last)],
                deg_out.at[pl.ds(pl.multiple_of(base + (NS - 1) * cw, 8),
                                 last)])

    return deg_kernel


def _make_msg_kernel(n, f, half, nwin):
    cw_r = _round8((half + NS - 1) // NS)
    last_r = half - (NS - 1) * cw_r
    t_edges = nwin * VSZ

    @functools.partial(
        pl.kernel,
        out_type=jax.ShapeDtypeStruct((n, f), jnp.float32),
        mesh=_sc_mesh(),
        compiler_params=pltpu.CompilerParams(use_tc_tiling_on_sc=False),
        scratch_types=[
            pltpu.VMEM((NB, VSZ), jnp.int32),
            pltpu.VMEM((NB, VSZ), jnp.int32),
            pltpu.VMEM((NB, VSZ), jnp.int32),
            pltpu.VMEM((NB * VSZ, f), jnp.float32),
            pltpu.VMEM_SHARED((half + DUM, f), jnp.float32),
            pltpu.SemaphoreType.DMA((NB,)),
            pltpu.SemaphoreType.DMA((NB,)),
            pltpu.SemaphoreType.DMA((NB,)),
        ],
    )
    def msg_kernel(g_hbm, s1_hbm, d1_hbm, acc_out,
                   s_win, d_win, offs, rows, acc_sh, isem, gsem, ssem):
        cid = lax.axis_index("c")
        sid = lax.axis_index("s")
        base = cid * half

        def rslot(b):
            return rows.at[pl.ds(b * VSZ, VSZ)]

        def stage_rows(off0, cnt, to_spmem):
            blk = NB * VSZ
            nfull, rem = divmod(cnt, blk)

            def body(q, _):
                off = pl.multiple_of(off0 + q * blk, 8)
                hoff = pl.multiple_of(base + off, 8)
                if to_spmem:
                    pltpu.sync_copy(g_hbm.at[pl.ds(hoff, blk)], rows)
                    pltpu.sync_copy(rows, acc_sh.at[pl.ds(off, blk)])
                else:
                    pltpu.sync_copy(acc_sh.at[pl.ds(off, blk)], rows)
                    pltpu.sync_copy(rows, acc_out.at[pl.ds(hoff, blk)])
                return 0

            lax.fori_loop(0, nfull, body, 0)
            if rem:
                off = pl.multiple_of(off0 + nfull * blk, 8)
                hoff = pl.multiple_of(base + off, 8)
                if to_spmem:
                    pltpu.sync_copy(g_hbm.at[pl.ds(hoff, rem)],
                                    rows.at[pl.ds(0, rem)])
                    pltpu.sync_copy(rows.at[pl.ds(0, rem)],
                                    acc_sh.at[pl.ds(off, rem)])
                else:
                    pltpu.sync_copy(acc_sh.at[pl.ds(off, rem)],
                                    rows.at[pl.ds(0, rem)])
                    pltpu.sync_copy(rows.at[pl.ds(0, rem)],
                                    acc_out.at[pl.ds(hoff, rem)])

        @pl.when(sid < NS - 1)
        def _():
            stage_rows(sid * cw_r, cw_r, True)

        @pl.when(sid == NS - 1)
        def _():
            stage_rows((NS - 1) * cw_r, last_r, True)

        plsc.subcore_barrier()

        def e0_of(v):
            return pl.multiple_of(sid * t_edges + v * VSZ, 8)

        def stage_idx_sync(v, b):
            pltpu.sync_copy(s1_hbm.at[pl.ds(e0_of(v), VSZ)], s_win.at[b])
            pltpu.sync_copy(d1_hbm.at[pl.ds(e0_of(v), VSZ)], d_win.at[b])

        def wait_isem(b):
            pltpu.make_async_copy(s1_hbm.at[pl.ds(0, VSZ)], s_win.at[b],
                                  isem.at[b]).wait()
            pltpu.make_async_copy(d1_hbm.at[pl.ds(0, VSZ)], d_win.at[b],
                                  isem.at[b]).wait()

        def wait_ssem(b):
            pltpu.make_async_copy(rslot(b), acc_sh.at[offs.at[b]],
                                  ssem.at[b]).wait()

        for b in range(NB):
            stage_idx_sync(b, b)
        for b in range(NB - 1):
            _offsets_chunk(d_win, offs, b, base, half)
            pltpu.async_copy(g_hbm.at[s_win.at[b]], rslot(b), gsem.at[b])

        def win_body(w, _):
            for b in range(NB):
                v = w * NB + b
                v2 = v + NB - 1
                b2 = (NB - 1 + b) % NB

                @pl.when(v2 < nwin)
                def _():
                    @pl.when(v >= 1)
                    def _():
                        wait_ssem(b2)
                        wait_isem(b2)

                    _offsets_chunk(d_win, offs, b2, base, half)
                    pltpu.async_copy(g_hbm.at[s_win.at[b2]], rslot(b2),
                                     gsem.at[b2])

                pltpu.make_async_copy(g_hbm.at[s_win.at[b]], rslot(b),
                                      gsem.at[b]).wait()
                pltpu.async_copy(rslot(b), acc_sh.at[offs.at[b]], ssem.at[b],
                                 add=True)

                @pl.when(v + NB < nwin)
                def _():
                    e0 = e0_of(v + NB)
                    pltpu.async_copy(s1_hbm.at[pl.ds(e0, VSZ)], s_win.at[b],
                                     isem.at[b])
                    pltpu.async_copy(d1_hbm.at[pl.ds(e0, VSZ)], d_win.at[b],
                                     isem.at[b])
            return 0

        lax.fori_loop(0, nwin // NB, win_body, 0)
        for b in range(NB):
            wait_ssem(b)
        plsc.subcore_barrier()

        @pl.when(sid < NS - 1)
        def _():
            stage_rows(sid * cw_r, cw_r, False)

        @pl.when(sid == NS - 1)
        def _():
            stage_rows((NS - 1) * cw_r, last_r, False)

    return msg_kernel


BN = 2000


def _tc_call(body, n, in_specs, out_specs, out_shapes):
    return pl.pallas_call(
        body,
        grid=(n // BN,),
        in_specs=in_specs,
        out_specs=out_specs,
        out_shape=out_shapes,
    )


def _row_spec(f):
    return pl.BlockSpec((BN, f), lambda i: (i, 0))


def _full_spec(a, b):
    return pl.BlockSpec((a, b), lambda i: (0, 0))


def _dense1_body(deg_ref, x_ref, w1_ref, g1_ref, dinv_ref):
    dinv = lax.rsqrt(deg_ref[...] + 1.0)
    g1_ref[...] = dinv * jnp.dot(x_ref[...], w1_ref[...],
                                 preferred_element_type=jnp.float32)
    dinv_ref[...] = dinv


def _dense2_body(acc_ref, dinv_ref, b1_ref, w2_ref, g2_ref):
    dinv = dinv_ref[...]
    h = jnp.maximum(dinv * acc_ref[...] + b1_ref[...], 0.0)
    g2_ref[...] = dinv * jnp.dot(h, w2_ref[...],
                                 preferred_element_type=jnp.float32)


def _dense3_body(acc_ref, dinv_ref, b2_ref, wf1_ref, bf1_ref, wf2_ref,
                 bf2_ref, out_ref):
    h = jnp.maximum(dinv_ref[...] * acc_ref[...] + b2_ref[...], 0.0)
    h = jnp.maximum(jnp.dot(h, wf1_ref[...],
                            preferred_element_type=jnp.float32) + bf1_ref[...],
                    0.0)
    out_ref[...] = jnp.dot(h, wf2_ref[...],
                           preferred_element_type=jnp.float32) + bf2_ref[...]


def kernel(x, edge_index, W1, b1, W2, b2, Wf1, bf1, Wf2, bf2):
    n, fin = x.shape
    e = edge_index.shape[1]
    half = n // 2
    f1 = W1.shape[1]
    f2 = W2.shape[1]
    ff1 = Wf1.shape[1]
    fout = Wf2.shape[1]

    per_round = NS * VSZ
    nwin = -(-e // per_round)
    nwin = -(-nwin // NB) * NB
    e_pad = nwin * per_round
    pad = e_pad - e
    s1 = jnp.concatenate(
        [edge_index[0], jnp.arange(pad, dtype=jnp.int32) % n])
    d1 = jnp.concatenate(
        [edge_index[1], jnp.full((pad,), -1, jnp.int32)])

    deg = _make_deg_kernel(n, half, nwin)(d1)
    deg2 = deg.reshape(n, 1)

    msg = _make_msg_kernel(n, f1, half, nwin)

    g1, dinv = _tc_call(
        _dense1_body, n,
        [_row_spec(1), _row_spec(fin), _full_spec(fin, f1)],
        [_row_spec(f1), _row_spec(1)],
        [jax.ShapeDtypeStruct((n, f1), jnp.float32),
         jax.ShapeDtypeStruct((n, 1), jnp.float32)],
    )(deg2, x, W1)

    acc1 = msg(g1, s1, d1)

    g2 = _tc_call(
        _dense2_body, n,
        [_row_spec(f1), _row_spec(1), _full_spec(1, f1), _full_spec(f1, f2)],
        _row_spec(f2),
        jax.ShapeDtypeStruct((n, f2), jnp.float32),
    )(acc1, dinv, b1.reshape(1, f1), W2)

    acc2 = msg(g2, s1, d1)

    out = _tc_call(
        _dense3_body, n,
        [_row_spec(f2), _row_spec(1), _full_spec(1, f2), _full_spec(f2, ff1),
         _full_spec(1, ff1), _full_spec(ff1, fout), _full_spec(1, fout)],
        _row_spec(fout),
        jax.ShapeDtypeStruct((n, fout), jnp.float32),
    )(acc2, dinv, b2.reshape(1, f2), Wf1, bf1.reshape(1, ff1), Wf2,
      bf2.reshape(1, fout))

    return out


# speedup vs baseline: 44.7011x; 1.2069x over previous
import functools

import jax
import jax.numpy as jnp
from jax import lax
from jax.experimental import pallas as pl
from jax.experimental.pallas import tpu as pltpu
from jax.experimental.pallas import tpu_sc as plsc

NC = 2
NS = 16
LANES = 16
VSZ = 256
NB = 3
DUM = 128


def _round8(v):
    return (v + 7) // 8 * 8


def _sc_mesh():
    return plsc.VectorSubcoreMesh(core_axis_name="c", subcore_axis_name="s")


def _offsets_chunk(d_win, offs, b, base, half):
    lane = lax.iota(jnp.int32, LANES)
    for k in range(VSZ // LANES):
        dd = d_win[b, pl.ds(k * LANES, LANES)]
        loc = dd - base
        inr = (dd >= base) & (loc < half)
        dum = half + (k % 8) * LANES + lane
        offs[b, pl.ds(k * LANES, LANES)] = jnp.where(inr, loc, dum)


def _make_deg_kernel(n, half, nwin):
    sh = _round8((half + DUM + NS - 1) // NS) * NS
    zb = sh // NS
    cw = _round8((half + NS - 1) // NS)
    last = half - (NS - 1) * cw
    t_edges = nwin * VSZ

    @functools.partial(
        pl.kernel,
        out_type=jax.ShapeDtypeStruct((n,), jnp.float32),
        mesh=_sc_mesh(),
        compiler_params=pltpu.CompilerParams(use_tc_tiling_on_sc=False),
        scratch_types=[
            pltpu.VMEM((NB, VSZ), jnp.int32),
            pltpu.VMEM((NB, VSZ), jnp.int32),
            pltpu.VMEM((VSZ,), jnp.float32),
            pltpu.VMEM((zb,), jnp.float32),
            pltpu.VMEM_SHARED((sh,), jnp.float32),
            pltpu.SemaphoreType.DMA((NB,)),
            pltpu.SemaphoreType.DMA((NB,)),
        ],
    )
    def deg_kernel(d1_hbm, deg_out, d_win, offs, ones, zbuf, deg_sh,
                   isem, ssem):
        cid = lax.axis_index("c")
        sid = lax.axis_index("s")
        base = cid * half

        def z_body(i, _):
            zbuf[pl.ds(i * LANES, LANES)] = jnp.zeros((LANES,), jnp.float32)
            return 0

        lax.fori_loop(0, zb // LANES, z_body, 0)

        def o_body(i, _):
            ones[pl.ds(i * LANES, LANES)] = jnp.ones((LANES,), jnp.float32)
            return 0

        lax.fori_loop(0, VSZ // LANES, o_body, 0)
        pltpu.sync_copy(zbuf, deg_sh.at[pl.ds(pl.multiple_of(sid * zb, 8),
                                              zb)])
        plsc.subcore_barrier()

        def e0_of(v):
            return pl.multiple_of(sid * t_edges + v * VSZ, 8)

        for b in range(NB):
            pltpu.sync_copy(d1_hbm.at[pl.ds(e0_of(b), VSZ)], d_win.at[b])

        def win_body(w, _):
            for b in range(NB):
                v = w * NB + b

                @pl.when(v >= NB)
                def _():
                    pltpu.make_async_copy(d1_hbm.at[pl.ds(0, VSZ)],
                                          d_win.at[b], isem.at[b]).wait()
                    pltpu.make_async_copy(ones, deg_sh.at[offs.at[b]],
                                          ssem.at[b]).wait()

                _offsets_chunk(d_win, offs, b, base, half)
                pltpu.async_copy(ones, deg_sh.at[offs.at[b]], ssem.at[b],
                                 add=True)

                @pl.when(v + NB < nwin)
                def _():
                    pltpu.async_copy(d1_hbm.at[pl.ds(e0_of(v + NB), VSZ)],
                                     d_win.at[b], isem.at[b])
            return 0

        lax.fori_loop(0, nwin // NB, win_body, 0)
        for b in range(NB):
            pltpu.make_async_copy(ones, deg_sh.at[offs.at[b]],
                                  ssem.at[b]).wait()
        plsc.subcore_barrier()

        @pl.when(sid < NS - 1)
        def _():
            off = pl.multiple_of(sid * cw, 8)
            pltpu.sync_copy(deg_sh.at[pl.ds(off, cw)], zbuf.at[pl.ds(0, cw)])
            pltpu.sync_copy(zbuf.at[pl.ds(0, cw)],
                            deg_out.at[pl.ds(pl.multiple_of(base + off, 8),
                                             cw)])

        @pl.when(sid == NS - 1)
        def _():
            pltpu.sync_copy(deg_sh.at[pl.ds((NS - 1) * cw, last)],
                            zbuf.at[pl.ds(0, last)])
            pltpu.sync_copy(
                zbuf.at[pl.ds(0, last)],
                deg_out.at[pl.ds(pl.multiple_of(base + (NS - 1) * cw, 8),
                                 last)])

    return deg_kernel


def _make_msg_kernel(n, f, half, nwin):
    cw_r = _round8((half + NS - 1) // NS)
    last_r = half - (NS - 1) * cw_r
    t_edges = nwin * VSZ

    @functools.partial(
        pl.kernel,
        out_type=jax.ShapeDtypeStruct((n, f), jnp.float32),
        mesh=_sc_mesh(),
        compiler_params=pltpu.CompilerParams(use_tc_tiling_on_sc=False),
        scratch_types=[
            pltpu.VMEM((NB, VSZ), jnp.int32),
            pltpu.VMEM((NB, VSZ), jnp.int32),
            pltpu.VMEM((NB, VSZ), jnp.int32),
            pltpu.VMEM((NB * VSZ, f), jnp.float32),
            pltpu.VMEM_SHARED((half + DUM, f), jnp.float32),
            pltpu.SemaphoreType.DMA((NB,)),
            pltpu.SemaphoreType.DMA((NB,)),
            pltpu.SemaphoreType.DMA((NB,)),
        ],
    )
    def msg_kernel(g_hbm, s1_hbm, d1_hbm, acc_out,
                   s_win, d_win, offs, rows, acc_sh, isem, gsem, ssem):
        cid = lax.axis_index("c")
        sid = lax.axis_index("s")
        base = cid * half

        def rslot(b):
            return rows.at[pl.ds(b * VSZ, VSZ)]

        def stage_rows(off0, cnt, to_spmem):
            blk = NB * VSZ
            nfull, rem = divmod(cnt, blk)

            def body(q, _):
                off = pl.multiple_of(off0 + q * blk, 8)
                hoff = pl.multiple_of(base + off, 8)
                if to_spmem:
                    pltpu.sync_copy(g_hbm.at[pl.ds(hoff, blk)], rows)
                    pltpu.sync_copy(rows, acc_sh.at[pl.ds(off, blk)])
                else:
                    pltpu.sync_copy(acc_sh.at[pl.ds(off, blk)], rows)
                    pltpu.sync_copy(rows, acc_out.at[pl.ds(hoff, blk)])
                return 0

            lax.fori_loop(0, nfull, body, 0)
            if rem:
                off = pl.multiple_of(off0 + nfull * blk, 8)
                hoff = pl.multiple_of(base + off, 8)
                if to_spmem:
                    pltpu.sync_copy(g_hbm.at[pl.ds(hoff, rem)],
                                    rows.at[pl.ds(0, rem)])
                    pltpu.sync_copy(rows.at[pl.ds(0, rem)],
                                    acc_sh.at[pl.ds(off, rem)])
                else:
                    pltpu.sync_copy(acc_sh.at[pl.ds(off, rem)],
                                    rows.at[pl.ds(0, rem)])
                    pltpu.sync_copy(rows.at[pl.ds(0, rem)],
                                    acc_out.at[pl.ds(hoff, rem)])

        @pl.when(sid < NS - 1)
        def _():
            stage_rows(sid * cw_r, cw_r, True)

        @pl.when(sid == NS - 1)
        def _():
            stage_rows((NS - 1) * cw_r, last_r, True)

        plsc.subcore_barrier()

        def e0_of(v):
            return pl.multiple_of(sid * t_edges + v * VSZ, 8)

        def stage_idx_sync(v, b):
            pltpu.sync_copy(s1_hbm.at[pl.ds(e0_of(v), VSZ)], s_win.at[b])
            pltpu.sync_copy(d1_hbm.at[pl.ds(e0_of(v), VSZ)], d_win.at[b])

        def wait_isem(b):
            pltpu.make_async_copy(s1_hbm.at[pl.ds(0, VSZ)], s_win.at[b],
                                  isem.at[b]).wait()
            pltpu.make_async_copy(d1_hbm.at[pl.ds(0, VSZ)], d_win.at[b],
                                  isem.at[b]).wait()

        def wait_ssem(b):
            pltpu.make_async_copy(rslot(b), acc_sh.at[offs.at[b]],
                                  ssem.at[b]).wait()

        for b in range(NB):
            stage_idx_sync(b, b)
        for b in range(NB - 1):
            _offsets_chunk(d_win, offs, b, base, half)
            pltpu.async_copy(g_hbm.at[s_win.at[b]], rslot(b), gsem.at[b])

        def win_body(w, _):
            for b in range(NB):
                v = w * NB + b
                v2 = v + NB - 1
                b2 = (NB - 1 + b) % NB

                @pl.when(v2 < nwin)
                def _():
                    @pl.when(v >= 1)
                    def _():
                        wait_ssem(b2)
                        wait_isem(b2)

                    _offsets_chunk(d_win, offs, b2, base, half)
                    pltpu.async_copy(g_hbm.at[s_win.at[b2]], rslot(b2),
                                     gsem.at[b2])

                pltpu.make_async_copy(g_hbm.at[s_win.at[b]], rslot(b),
                                      gsem.at[b]).wait()
                pltpu.async_copy(rslot(b), acc_sh.at[offs.at[b]], ssem.at[b],
                                 add=True)

                @pl.when(v + NB < nwin)
                def _():
                    e0 = e0_of(v + NB)
                    pltpu.async_copy(s1_hbm.at[pl.ds(e0, VSZ)], s_win.at[b],
                                     isem.at[b])
                    pltpu.async_copy(d1_hbm.at[pl.ds(e0, VSZ)], d_win.at[b],
                                     isem.at[b])
            return 0

        lax.fori_loop(0, nwin // NB, win_body, 0)
        for b in range(NB):
            wait_ssem(b)
        plsc.subcore_barrier()

        @pl.when(sid < NS - 1)
        def _():
            stage_rows(sid * cw_r, cw_r, False)

        @pl.when(sid == NS - 1)
        def _():
            stage_rows((NS - 1) * cw_r, last_r, False)

    return msg_kernel


BF = 1000


def _tc_call(body, nf, in_specs, out_specs, out_shapes):
    return pl.pallas_call(
        body,
        grid=(nf // BF,),
        in_specs=in_specs,
        out_specs=out_specs,
        out_shape=out_shapes,
    )


def _blk_spec(c):
    return pl.BlockSpec((BF, c), lambda i: (i, 0))


def _full_spec(a, b):
    return pl.BlockSpec((a, b), lambda i: (0, 0))


def _dense1_body(degf_ref, xf_ref, k1_ref, g1_ref, dinvf_ref):
    dinv = lax.rsqrt(degf_ref[...] + 1.0)
    g1_ref[...] = dinv * jnp.dot(xf_ref[...], k1_ref[...],
                                 preferred_element_type=jnp.float32)
    dinvf_ref[...] = dinv


def _dense2_body(accf_ref, dinvf_ref, b1f_ref, k2_ref, g2f_ref):
    dinv = dinvf_ref[...]
    h = jnp.maximum(dinv * accf_ref[...] + b1f_ref[...], 0.0)
    g2f_ref[...] = dinv * jnp.dot(h, k2_ref[...],
                                  preferred_element_type=jnp.float32)


def _dense3_body(accf_ref, dinvf_ref, b2f_ref, k31_ref, bf1f_ref, k32_ref,
                 bf2f_ref, outf_ref):
    h = jnp.maximum(dinvf_ref[...] * accf_ref[...] + b2f_ref[...], 0.0)
    h = jnp.maximum(jnp.dot(h, k31_ref[...],
                            preferred_element_type=jnp.float32)
                    + bf1f_ref[...], 0.0)
    outf_ref[...] = jnp.dot(h, k32_ref[...],
                            preferred_element_type=jnp.float32) + bf2f_ref[...]


def kernel(x, edge_index, W1, b1, W2, b2, Wf1, bf1, Wf2, bf2):
    n, fin = x.shape
    e = edge_index.shape[1]
    half = n // 2
    f1 = W1.shape[1]
    f2 = W2.shape[1]
    ff1 = Wf1.shape[1]
    fout = Wf2.shape[1]
    npr = 128 // f1
    nf = n * f1 // 128
    eye = jnp.eye(npr, dtype=jnp.float32)

    per_round = NS * VSZ
    nwin = -(-e // per_round)
    nwin = -(-nwin // NB) * NB
    e_pad = nwin * per_round
    pad = e_pad - e
    s1 = jnp.concatenate(
        [edge_index[0], jnp.arange(pad, dtype=jnp.int32) % n])
    d1 = jnp.concatenate(
        [edge_index[1], jnp.full((pad,), -1, jnp.int32)])

    deg = _make_deg_kernel(n, half, nwin)(d1)
    degf = jnp.repeat(deg, f1).reshape(nf, 128)

    xf = jnp.concatenate(
        [x, jnp.zeros((n, f1 - fin), jnp.float32)], axis=1).reshape(nf, 128)
    w1p = jnp.zeros((f1, f1), jnp.float32).at[:fin].set(W1)
    k1 = jnp.kron(eye, w1p)
    k2 = jnp.kron(eye, W2)
    k31 = jnp.kron(eye, Wf1)
    k32 = jnp.kron(eye, Wf2)
    b1f = jnp.tile(b1, npr).reshape(1, 128)
    b2f = jnp.tile(b2, npr).reshape(1, 128)
    bf1f = jnp.tile(bf1, npr).reshape(1, npr * ff1)
    bf2f = jnp.tile(bf2, npr).reshape(1, npr * fout)

    msg = _make_msg_kernel(n, f1, half, nwin)

    g1f, dinvf = _tc_call(
        _dense1_body, nf,
        [_blk_spec(128), _blk_spec(128), _full_spec(128, 128)],
        [_blk_spec(128), _blk_spec(128)],
        [jax.ShapeDtypeStruct((nf, 128), jnp.float32),
         jax.ShapeDtypeStruct((nf, 128), jnp.float32)],
    )(degf, xf, k1)

    acc1 = msg(g1f.reshape(n, f1), s1, d1)

    g2f = _tc_call(
        _dense2_body, nf,
        [_blk_spec(128), _blk_spec(128), _full_spec(1, 128),
         _full_spec(128, 128)],
        _blk_spec(128),
        jax.ShapeDtypeStruct((nf, 128), jnp.float32),
    )(acc1.reshape(nf, 128), dinvf, b1f, k2)

    acc2 = msg(g2f.reshape(n, f1), s1, d1)

    outf = _tc_call(
        _dense3_body, nf,
        [_blk_spec(128), _blk_spec(128), _full_spec(1, 128),
         _full_spec(128, npr * ff1), _full_spec(1, npr * ff1),
         _full_spec(npr * ff1, npr * fout), _full_spec(1, npr * fout)],
        pl.BlockSpec((BF, npr * fout), lambda i: (i, 0)),
        jax.ShapeDtypeStruct((nf, npr * fout), jnp.float32),
    )(acc2.reshape(nf, 128), dinvf, b2f, k31, bf1f, k32, bf2f)

    return outf.reshape(n, fout)


# speedup vs baseline: 56.9388x; 1.2738x over previous
import functools

import jax
import jax.numpy as jnp
from jax import lax
from jax.experimental import pallas as pl
from jax.experimental.pallas import tpu as pltpu
from jax.experimental.pallas import tpu_sc as plsc

NC = 2
NS = 16
LANES = 16
VSZ = 384
NB = 3
DUM = 64
FH = 16
NPAD = 102400


def _sc_mesh():
    return plsc.VectorSubcoreMesh(core_axis_name="c", subcore_axis_name="s")


def _offsets_chunk(d_win, offs, b):
    lane = lax.iota(jnp.int32, LANES)
    for k in range(VSZ // LANES):
        dd = d_win[b, pl.ds(k * LANES, LANES)]
        dum = NPAD + (k % 4) * LANES + lane
        offs[b, pl.ds(k * LANES, LANES)] = jnp.where(dd >= 0, dd, dum)


def _adjust_src(s_win, b, cbase):
    for k in range(VSZ // LANES):
        s_win[b, pl.ds(k * LANES, LANES)] = (
            s_win[b, pl.ds(k * LANES, LANES)] + cbase)


def _make_deg_kernel(nwin):
    sh = NPAD + 128
    zb = sh // NS
    rw = NPAD // NS
    t_edges = nwin * VSZ

    @functools.partial(
        pl.kernel,
        out_type=jax.ShapeDtypeStruct((2 * NPAD,), jnp.float32),
        mesh=_sc_mesh(),
        compiler_params=pltpu.CompilerParams(use_tc_tiling_on_sc=False),
        scratch_types=[
            pltpu.VMEM((NB, VSZ), jnp.int32),
            pltpu.VMEM((NB, VSZ), jnp.int32),
            pltpu.VMEM((VSZ,), jnp.float32),
            pltpu.VMEM((zb,), jnp.float32),
            pltpu.VMEM_SHARED((sh,), jnp.float32),
            pltpu.SemaphoreType.DMA((NB,)),
            pltpu.SemaphoreType.DMA((NB,)),
        ],
    )
    def deg_kernel(d1_hbm, deg_out, d_win, offs, ones, zbuf, deg_sh,
                   isem, ssem):
        cid = lax.axis_index("c")
        sid = lax.axis_index("s")

        def z_body(i, _):
            zbuf[pl.ds(i * LANES, LANES)] = jnp.zeros((LANES,), jnp.float32)
            return 0

        lax.fori_loop(0, zb // LANES, z_body, 0)

        def o_body(i, _):
            ones[pl.ds(i * LANES, LANES)] = jnp.ones((LANES,), jnp.float32)
            return 0

        lax.fori_loop(0, VSZ // LANES, o_body, 0)
        pltpu.sync_copy(zbuf, deg_sh.at[pl.ds(pl.multiple_of(sid * zb, 8),
                                              zb)])
        plsc.subcore_barrier()

        def e0_of(v):
            return pl.multiple_of(
                (cid * NS + sid) * t_edges + v * VSZ, 8)

        for b in range(NB):
            pltpu.sync_copy(d1_hbm.at[pl.ds(e0_of(b), VSZ)], d_win.at[b])

        def win_body(w, _):
            for b in range(NB):
                v = w * NB + b

                @pl.when(v >= NB)
                def _():
                    pltpu.make_async_copy(d1_hbm.at[pl.ds(0, VSZ)],
                                          d_win.at[b], isem.at[b]).wait()
                    pltpu.make_async_copy(ones, deg_sh.at[offs.at[b]],
                                          ssem.at[b]).wait()

                _offsets_chunk(d_win, offs, b)
                pltpu.async_copy(ones, deg_sh.at[offs.at[b]], ssem.at[b],
                                 add=True)

                @pl.when(v + NB < nwin)
                def _():
                    pltpu.async_copy(d1_hbm.at[pl.ds(e0_of(v + NB), VSZ)],
                                     d_win.at[b], isem.at[b])
            return 0

        lax.fori_loop(0, nwin // NB, win_body, 0)
        for b in range(NB):
            pltpu.make_async_copy(ones, deg_sh.at[offs.at[b]],
                                  ssem.at[b]).wait()
        plsc.subcore_barrier()

        off = pl.multiple_of(sid * rw, 8)
        pltpu.sync_copy(deg_sh.at[pl.ds(off, rw)], zbuf.at[pl.ds(0, rw)])
        pltpu.sync_copy(zbuf.at[pl.ds(0, rw)],
                        deg_out.at[pl.ds(pl.multiple_of(cid * NPAD + off, 8),
                                         rw)])

    return deg_kernel


def _make_msg_kernel(nwin):
    rw = NPAD // NS
    t_edges = nwin * VSZ

    @functools.partial(
        pl.kernel,
        out_type=jax.ShapeDtypeStruct((2 * NPAD, FH), jnp.float32),
        mesh=_sc_mesh(),
        compiler_params=pltpu.CompilerParams(use_tc_tiling_on_sc=False),
        scratch_types=[
            pltpu.VMEM((NB, VSZ), jnp.int32),
            pltpu.VMEM((NB, VSZ), jnp.int32),
            pltpu.VMEM((NB, VSZ), jnp.int32),
            pltpu.VMEM((NB * VSZ, FH), jnp.float32),
            pltpu.VMEM_SHARED((NPAD + DUM, FH), jnp.float32),
            pltpu.SemaphoreType.DMA((NB,)),
            pltpu.SemaphoreType.DMA((NB,)),
            pltpu.SemaphoreType.DMA((NB,)),
        ],
    )
    def msg_kernel(g_hbm, s1_hbm, d1_hbm, acc_out,
                   s_win, d_win, offs, rows, acc_sh, isem, gsem, ssem):
        cid = lax.axis_index("c")
        sid = lax.axis_index("s")
        cbase = cid * NPAD

        def rslot(b):
            return rows.at[pl.ds(b * VSZ, VSZ)]

        def stage_rows(to_spmem):
            blk = NB * VSZ
            nfull, rem = divmod(rw, blk)

            def body(q, _):
                off = pl.multiple_of(sid * rw + q * blk, 8)
                hoff = pl.multiple_of(cbase + off, 8)
                if to_spmem:
                    pltpu.sync_copy(g_hbm.at[pl.ds(hoff, blk)], rows)
                    pltpu.sync_copy(rows, acc_sh.at[pl.ds(off, blk)])
                else:
                    pltpu.sync_copy(acc_sh.at[pl.ds(off, blk)], rows)
                    pltpu.sync_copy(rows, acc_out.at[pl.ds(hoff, blk)])
                return 0

            lax.fori_loop(0, nfull, body, 0)
            if rem:
                off = pl.multiple_of(sid * rw + nfull * blk, 8)
                hoff = pl.multiple_of(cbase + off, 8)
                if to_spmem:
                    pltpu.sync_copy(g_hbm.at[pl.ds(hoff, rem)],
                                    rows.at[pl.ds(0, rem)])
                    pltpu.sync_copy(rows.at[pl.ds(0, rem)],
                                    acc_sh.at[pl.ds(off, rem)])
                else:
                    pltpu.sync_copy(acc_sh.at[pl.ds(off, rem)],
                                    rows.at[pl.ds(0, rem)])
                    pltpu.sync_copy(rows.at[pl.ds(0, rem)],
                                    acc_out.at[pl.ds(hoff, rem)])

        stage_rows(True)
        plsc.subcore_barrier()

        def e0_of(v):
            return pl.multiple_of(sid * t_edges + v * VSZ, 8)

        def wait_isem(b):
            pltpu.make_async_copy(s1_hbm.at[pl.ds(0, VSZ)], s_win.at[b],
                                  isem.at[b]).wait()
            pltpu.make_async_copy(d1_hbm.at[pl.ds(0, VSZ)], d_win.at[b],
                                  isem.at[b]).wait()

        def wait_ssem(b):
            pltpu.make_async_copy(rslot(b), acc_sh.at[offs.at[b]],
                                  ssem.at[b]).wait()

        for b in range(NB):
            pltpu.sync_copy(s1_hbm.at[pl.ds(e0_of(b), VSZ)], s_win.at[b])
            pltpu.sync_copy(d1_hbm.at[pl.ds(e0_of(b), VSZ)], d_win.at[b])
        for b in range(NB - 1):
            _adjust_src(s_win, b, cbase)
            _offsets_chunk(d_win, offs, b)
            pltpu.async_copy(g_hbm.at[s_win.at[b]], rslot(b), gsem.at[b])

        def win_body(w, _):
            for b in range(NB):
                v = w * NB + b
                v2 = v + NB - 1
                b2 = (NB - 1 + b) % NB

                @pl.when(v2 < nwin)
                def _():
                    @pl.when(v >= 1)
                    def _():
                        wait_ssem(b2)
                        wait_isem(b2)

                    _adjust_src(s_win, b2, cbase)
                    _offsets_chunk(d_win, offs, b2)
                    pltpu.async_copy(g_hbm.at[s_win.at[b2]], rslot(b2),
                                     gsem.at[b2])

                pltpu.make_async_copy(g_hbm.at[s_win.at[b]], rslot(b),
                                      gsem.at[b]).wait()
                pltpu.async_copy(rslot(b), acc_sh.at[offs.at[b]], ssem.at[b],
                                 add=True)

                @pl.when(v + NB < nwin)
                def _():
                    e0 = e0_of(v + NB)
                    pltpu.async_copy(s1_hbm.at[pl.ds(e0, VSZ)], s_win.at[b],
                                     isem.at[b])
                    pltpu.async_copy(d1_hbm.at[pl.ds(e0, VSZ)], d_win.at[b],
                                     isem.at[b])
            return 0

        lax.fori_loop(0, nwin // NB, win_body, 0)
        for b in range(NB):
            wait_ssem(b)
        plsc.subcore_barrier()
        stage_rows(False)

    return msg_kernel


NF16 = NPAD * FH // 128
BF = 800


def _dense1_body(degf_ref, x4_ref, pw_ref, g1_ref, dinvf_ref):
    dinv = lax.rsqrt(degf_ref[...] + 1.0)
    g1_ref[...] = dinv * jnp.dot(x4_ref[...], pw_ref[0],
                                 preferred_element_type=jnp.float32)
    dinvf_ref[...] = dinv


def _dense2_body(alo_ref, ahi_ref, dinvf_ref, b1lo_ref, b1hi_ref,
                 ka_ref, kb_ref, g2_ref):
    dinv = dinvf_ref[...]
    hlo = jnp.maximum(dinv * alo_ref[...] + b1lo_ref[...], 0.0)
    hhi = jnp.maximum(dinv * ahi_ref[...] + b1hi_ref[...], 0.0)
    g2_ref[...] = dinv * (
        jnp.dot(hlo, ka_ref[0], preferred_element_type=jnp.float32)
        + jnp.dot(hhi, kb_ref[0], preferred_element_type=jnp.float32))


def _dense3_body(alo_ref, ahi_ref, dinvf_ref, b2lo_ref, b2hi_ref,
                 kf1a_ref, kf1b_ref, bf1_ref, kf2_ref, bf2_ref, out_ref):
    dinv = dinvf_ref[...]
    hlo = jnp.maximum(dinv * alo_ref[...] + b2lo_ref[...], 0.0)
    hhi = jnp.maximum(dinv * ahi_ref[...] + b2hi_ref[...], 0.0)
    hf = jnp.maximum(
        jnp.dot(hlo, kf1a_ref[...], preferred_element_type=jnp.float32)
        + jnp.dot(hhi, kf1b_ref[...], preferred_element_type=jnp.float32)
        + bf1_ref[...], 0.0)
    out_ref[...] = jnp.dot(hf, kf2_ref[...],
                           preferred_element_type=jnp.float32) + bf2_ref[...]


def kernel(x, edge_index, W1, b1, W2, b2, Wf1, bf1, Wf2, bf2):
    n, fin = x.shape
    e = edge_index.shape[1]
    f1 = W1.shape[1]
    ff1 = Wf1.shape[1]
    fout = Wf2.shape[1]
    eye8 = jnp.eye(8, dtype=jnp.float32)
    kron = jnp.kron

    quantum = NC * NS * VSZ * NB
    e_pad = -(-e // quantum) * quantum
    pad = e_pad - e
    s1 = jnp.concatenate(
        [edge_index[0], jnp.arange(pad, dtype=jnp.int32) % n])
    d1 = jnp.concatenate(
        [edge_index[1], jnp.full((pad,), -1, jnp.int32)])
    nwin_deg = e_pad // (NC * NS * VSZ)
    nwin_msg = e_pad // (NS * VSZ)

    deg2 = _make_deg_kernel(nwin_deg)(d1)
    degf = jnp.repeat(deg2[:NPAD] + deg2[NPAD:], FH).reshape(NF16, 128)

    x4f = jnp.pad(x.reshape(-1), (0, NPAD * fin - n * fin)).reshape(
        NF16, 8 * fin)
    pw = jnp.stack([kron(eye8, W1[:, :FH]), kron(eye8, W1[:, FH:])])
    ka = jnp.stack([kron(eye8, W2[:FH, :FH]), kron(eye8, W2[:FH, FH:])])
    kb = jnp.stack([kron(eye8, W2[FH:, :FH]), kron(eye8, W2[FH:, FH:])])
    b1lo = jnp.tile(b1[:FH], 8).reshape(1, 128)
    b1hi = jnp.tile(b1[FH:], 8).reshape(1, 128)
    b2lo = jnp.tile(b2[:FH], 8).reshape(1, 128)
    b2hi = jnp.tile(b2[FH:], 8).reshape(1, 128)
    kf1a = kron(eye8, Wf1[:FH])
    kf1b = kron(eye8, Wf1[FH:])
    bf1t = jnp.tile(bf1, 8).reshape(1, 8 * ff1)
    kf2 = kron(eye8, Wf2)
    bf2t = jnp.tile(bf2, 8).reshape(1, 8 * fout)

    half_blk = pl.BlockSpec((BF, 128), lambda i: (i % 16, 0))
    lo_blk = half_blk
    hi_blk = pl.BlockSpec((BF, 128), lambda i: (16 + i % 16, 0))
    cat_blk = pl.BlockSpec((BF, 128), lambda i: (i, 0))
    w_blk = pl.BlockSpec((1, 32, 128), lambda i: (i // 16, 0, 0))
    k_blk = pl.BlockSpec((1, 128, 128), lambda i: (i // 16, 0, 0))
    x4_blk = pl.BlockSpec((BF, 8 * fin), lambda i: (i % 16, 0))

    def full2(a, b):
        return pl.BlockSpec((a, b), lambda i: (0, 0))

    g1f, dinvf = pl.pallas_call(
        _dense1_body,
        grid=(32,),
        in_specs=[half_blk, x4_blk, w_blk],
        out_specs=[cat_blk, half_blk],
        out_shape=[jax.ShapeDtypeStruct((2 * NF16, 128), jnp.float32),
                   jax.ShapeDtypeStruct((NF16, 128), jnp.float32)],
    )(degf, x4f, pw)

    msg = _make_msg_kernel(nwin_msg)

    acc1 = msg(g1f.reshape(2 * NPAD, FH), s1, d1)

    g2f = pl.pallas_call(
        _dense2_body,
        grid=(32,),
        in_specs=[lo_blk, hi_blk, half_blk, full2(1, 128), full2(1, 128),
                  k_blk, k_blk],
        out_specs=cat_blk,
        out_shape=jax.ShapeDtypeStruct((2 * NF16, 128), jnp.float32),
    )(acc1.reshape(2 * NF16, 128), acc1.reshape(2 * NF16, 128), dinvf,
      b1lo, b1hi, ka, kb)

    acc2 = msg(g2f.reshape(2 * NPAD, FH), s1, d1)

    outf = pl.pallas_call(
        _dense3_body,
        grid=(16,),
        in_specs=[pl.BlockSpec((BF, 128), lambda i: (i, 0)),
                  pl.BlockSpec((BF, 128), lambda i: (16 + i, 0)),
                  pl.BlockSpec((BF, 128), lambda i: (i, 0)),
                  full2(1, 128), full2(1, 128),
                  full2(128, 8 * ff1), full2(128, 8 * ff1),
                  full2(1, 8 * ff1), full2(8 * ff1, 8 * fout),
                  full2(1, 8 * fout)],
        out_specs=pl.BlockSpec((BF, 8 * fout), lambda i: (i, 0)),
        out_shape=jax.ShapeDtypeStruct((NF16, 8 * fout), jnp.float32),
    )(acc2.reshape(2 * NF16, 128), acc2.reshape(2 * NF16, 128), dinvf,
      b2lo, b2hi, kf1a, kf1b, bf1t, kf2, bf2t)

    return outf.reshape(-1)[:n * fout].reshape(n, fout)
